# baseline (speedup 1.0000x reference)
import functools

import jax
import jax.numpy as jnp
from jax.experimental import pallas as pl

_LOG_FACTOR = 6.9077542789816375


def _distnet_kernel(x_ref, p_ref, beta_ref, out_ref, *, n_points, block):
    j = pl.program_id(0)
    nb = pl.num_programs(0)
    xb = x_ref[...]
    pb = p_ref[...]

    base = j * block
    row = jax.lax.broadcasted_iota(jnp.int32, (block, 1), 0)
    pb = jnp.where(row + base < n_points, pb, pb[0:1, :])

    mm = jax.lax.dot_general(
        pb, xb, (((1,), (1,)), ((), ())),
        preferred_element_type=jnp.float32,
        precision=jax.lax.Precision.HIGHEST,
    )
    p2 = jnp.sum(pb * pb, axis=1, keepdims=True)
    cmin = jnp.min(p2 - 2.0 * mm, axis=0, keepdims=True)

    @pl.when(j == 0)
    def _init():
        out_ref[...] = cmin

    @pl.when(j > 0)
    def _acc():
        out_ref[...] = jnp.minimum(out_ref[...], cmin)

    @pl.when(j == nb - 1)
    def _final():
        w = xb * xb
        x2 = jax.lax.dot_general(
            jnp.ones((1, w.shape[1]), jnp.float32), w,
            (((1,), (1,)), ((), ())),
            preferred_element_type=jnp.float32,
            precision=jax.lax.Precision.HIGHEST,
        )
        d2 = jnp.maximum(x2 + out_ref[...], 0.0)
        b = jax.nn.softplus(beta_ref[...])
        alpha = -_LOG_FACTOR * b
        out_ref[...] = jax.nn.sigmoid((d2 + alpha) / b)


def kernel(x, points, beta):
    q, d = x.shape
    n, _ = points.shape
    block = 2048
    nb = pl.cdiv(n, block)
    out = pl.pallas_call(
        functools.partial(_distnet_kernel, n_points=n, block=block),
        grid=(nb,),
        in_specs=[
            pl.BlockSpec((q, d), lambda j: (0, 0)),
            pl.BlockSpec((block, d), lambda j: (j, 0)),
            pl.BlockSpec((1, 1), lambda j: (0, 0)),
        ],
        out_specs=pl.BlockSpec((1, q), lambda j: (0, 0)),
        out_shape=jax.ShapeDtypeStruct((1, q), jnp.float32),
    )(x, points, beta.reshape(1, 1))
    return out.reshape(q)


# speedup vs baseline: 3.6771x; 3.6771x over previous
import functools

import jax
import jax.numpy as jnp
from jax.experimental import pallas as pl

_LOG_FACTOR = 6.9077542789816375


def _distnet_kernel(x_ref, p_ref, beta_ref, out_ref, *, n_points, block):
    j = pl.program_id(0)
    nb = pl.num_programs(0)
    xb = x_ref[...]
    pb = p_ref[...]

    base = j * block
    row = jax.lax.broadcasted_iota(jnp.int32, (block, 1), 0)
    pb = jnp.where(row + base < n_points, pb, pb[0:1, :])

    pa = jnp.concatenate([pb, pb * pb], axis=1)
    xa = jnp.concatenate([-2.0 * xb, jnp.ones_like(xb)], axis=1)
    c = jax.lax.dot_general(
        pa, xa, (((1,), (1,)), ((), ())),
        preferred_element_type=jnp.float32,
    )
    cmin = jnp.min(c, axis=0, keepdims=True)

    @pl.when(j == 0)
    def _init():
        out_ref[...] = cmin

    @pl.when(j > 0)
    def _acc():
        out_ref[...] = jnp.minimum(out_ref[...], cmin)

    @pl.when(j == nb - 1)
    def _final():
        w = xb * xb
        x2 = jax.lax.dot_general(
            jnp.ones((1, w.shape[1]), jnp.float32), w,
            (((1,), (1,)), ((), ())),
            preferred_element_type=jnp.float32,
        )
        d2 = jnp.maximum(x2 + out_ref[...], 0.0)
        b = jax.nn.softplus(beta_ref[...])
        alpha = -_LOG_FACTOR * b
        out_ref[...] = jax.nn.sigmoid((d2 + alpha) / b)


def kernel(x, points, beta):
    q, d = x.shape
    n, _ = points.shape
    block = 2048
    nb = pl.cdiv(n, block)
    out = pl.pallas_call(
        functools.partial(_distnet_kernel, n_points=n, block=block),
        grid=(nb,),
        in_specs=[
            pl.BlockSpec((q, d), lambda j: (0, 0)),
            pl.BlockSpec((block, d), lambda j: (j, 0)),
            pl.BlockSpec((1, 1), lambda j: (0, 0)),
        ],
        out_specs=pl.BlockSpec((1, q), lambda j: (0, 0)),
        out_shape=jax.ShapeDtypeStruct((1, q), jnp.float32),
    )(x, points, beta.reshape(1, 1))
    return out.reshape(q)
